# baseline (speedup 1.0000x reference)
import functools

import jax
import jax.numpy as jnp
from jax import lax
from jax.experimental import pallas as pl
from jax.experimental.pallas import tpu as pltpu
from jax.experimental.pallas import tpu_sc as plsc

ROWS = 4096
COLS = 200
DIM = 4
NUM_EMB = 8

_info = plsc.get_sparse_core_info()
NC = _info.num_cores
NS = _info.num_subcores
NW = NC * NS
IBLK = ROWS // NW
JBLK = 8
NJB = COLS // JBLK


def _make_lookup():
    mesh = plsc.VectorSubcoreMesh(core_axis_name="c", subcore_axis_name="s")

    @functools.partial(
        pl.kernel,
        mesh=mesh,
        compiler_params=pltpu.CompilerParams(
            needs_layout_passes=False,
            use_tc_tiling_on_sc=False,
        ),
        out_type=jax.ShapeDtypeStruct((COLS, NW, DIM * IBLK), jnp.float32),
        scratch_types=[
            pltpu.VMEM((DIM * 16,), jnp.float32),
            pltpu.VMEM((NJB, JBLK * IBLK), jnp.int32),
            pltpu.VMEM((COLS, DIM * IBLK), jnp.float32),
            pltpu.SemaphoreType.DMA,
        ],
    )
    def lookup(ids_hbm, table_hbm, out_hbm, table_v, idx_v, out_v, sem):
        wid = lax.axis_index("s") * NC + lax.axis_index("c")
        pltpu.sync_copy(table_hbm, table_v)
        pltpu.sync_copy(ids_hbm.at[:, wid], idx_v)

        tvecs = [table_v[pl.ds(16 * d, 16)] for d in range(DIM)]

        def make_body(lo, hi):
            @plsc.parallel_loop(lo, hi, unroll=2)
            def body(j):
                jb = j // JBLK
                jm = (j % JBLK) * IBLK
                for k in range(IBLK // 16):
                    ids16 = idx_v[jb, pl.ds(jm + k * 16, 16)]
                    for d in range(DIM):
                        vals = tvecs[d].at[ids16].get(
                            mode="promise_in_bounds"
                        )
                        out_v[j, pl.ds(d * IBLK + k * 16, 16)] = vals

        nq = 4
        q = COLS // nq
        cps = []
        for i in range(nq - 1):
            make_body(i * q, (i + 1) * q)
            cps.append(
                pltpu.async_copy(
                    out_v.at[pl.ds(i * q, q)],
                    out_hbm.at[pl.ds(i * q, q), wid],
                    sem,
                )
            )
        make_body((nq - 1) * q, COLS)
        for cp in cps:
            cp.wait()
        pltpu.sync_copy(
            out_v.at[pl.ds((nq - 1) * q, COLS - (nq - 1) * q)],
            out_hbm.at[pl.ds((nq - 1) * q, COLS - (nq - 1) * q), wid],
        )

    return lookup


_lookup = _make_lookup()


def kernel(input_ids, embed_table):
    ids3 = (
        input_ids.astype(jnp.int32)
        .T.reshape(NJB, JBLK, NW, IBLK)
        .transpose(0, 2, 1, 3)
        .reshape(NJB, NW, JBLK * IBLK)
    )
    tpad = jnp.concatenate(
        [embed_table.T, jnp.zeros((DIM, 16 - NUM_EMB), jnp.float32)], axis=1
    ).reshape(-1)
    b = _lookup(ids3, tpad)
    return (
        b.reshape(COLS, NW, DIM, IBLK)
        .transpose(1, 3, 0, 2)
        .reshape(ROWS, COLS, DIM)
    )


# speedup vs baseline: 1.0083x; 1.0083x over previous
import functools

import jax
import jax.numpy as jnp
from jax import lax
from jax.experimental import pallas as pl
from jax.experimental.pallas import tpu as pltpu
from jax.experimental.pallas import tpu_sc as plsc

ROWS = 4096
COLS = 200
DIM = 4
NUM_EMB = 8

_info = plsc.get_sparse_core_info()
NC = _info.num_cores
NS = _info.num_subcores
NW = NC * NS
IBLK = ROWS // NW
JBLK = 8
NJB = COLS // JBLK


def _make_lookup():
    mesh = plsc.VectorSubcoreMesh(core_axis_name="c", subcore_axis_name="s")

    @functools.partial(
        pl.kernel,
        mesh=mesh,
        compiler_params=pltpu.CompilerParams(
            needs_layout_passes=False,
            use_tc_tiling_on_sc=False,
        ),
        out_type=jax.ShapeDtypeStruct((COLS, NW, DIM * IBLK), jnp.float32),
        scratch_types=[
            pltpu.VMEM((DIM * 16,), jnp.float32),
            pltpu.VMEM((NJB, JBLK * IBLK), jnp.int32),
            pltpu.VMEM((COLS, DIM * IBLK), jnp.float32),
            pltpu.SemaphoreType.DMA,
        ],
    )
    def lookup(ids_hbm, table_hbm, out_hbm, table_v, idx_v, out_v, sem):
        wid = lax.axis_index("s") * NC + lax.axis_index("c")
        pltpu.sync_copy(table_hbm, table_v)
        pltpu.sync_copy(ids_hbm.at[:, wid], idx_v)

        tvecs = [table_v[pl.ds(16 * d, 16)] for d in range(DIM)]
        half = COLS // 2

        def make_body(lo, hi):
            @plsc.parallel_loop(lo, hi, unroll=4)
            def body(j):
                jb = j // JBLK
                jm = (j % JBLK) * IBLK
                for k in range(IBLK // 16):
                    ids16 = idx_v[jb, pl.ds(jm + k * 16, 16)]
                    for d in range(DIM):
                        vals = tvecs[d].at[ids16].get(
                            mode="promise_in_bounds"
                        )
                        out_v[j, pl.ds(d * IBLK + k * 16, 16)] = vals

        make_body(0, half)
        cp = pltpu.async_copy(
            out_v.at[pl.ds(0, half)], out_hbm.at[pl.ds(0, half), wid], sem
        )
        make_body(half, COLS)
        cp.wait()
        pltpu.sync_copy(
            out_v.at[pl.ds(half, COLS - half)],
            out_hbm.at[pl.ds(half, COLS - half), wid],
        )

    return lookup


_lookup = _make_lookup()


def kernel(input_ids, embed_table):
    ids3 = (
        input_ids.astype(jnp.int32)
        .T.reshape(NJB, JBLK, NW, IBLK)
        .transpose(0, 2, 1, 3)
        .reshape(NJB, NW, JBLK * IBLK)
    )
    tpad = jnp.concatenate(
        [embed_table.T, jnp.zeros((DIM, 16 - NUM_EMB), jnp.float32)], axis=1
    ).reshape(-1)
    b = _lookup(ids3, tpad)
    return (
        b.reshape(COLS, NW, DIM, IBLK)
        .transpose(1, 3, 0, 2)
        .reshape(ROWS, COLS, DIM)
    )


# speedup vs baseline: 1.0327x; 1.0242x over previous
import functools

import jax
import jax.numpy as jnp
from jax import lax
from jax.experimental import pallas as pl
from jax.experimental.pallas import tpu as pltpu
from jax.experimental.pallas import tpu_sc as plsc

ROWS = 4096
COLS = 200
DIM = 4
NUM_EMB = 8

_info = plsc.get_sparse_core_info()
NC = _info.num_cores
NS = _info.num_subcores
NW = NC * NS
IBLK = ROWS // NW
JBLK = 8
NJB = COLS // JBLK


def _make_lookup():
    mesh = plsc.VectorSubcoreMesh(core_axis_name="c", subcore_axis_name="s")

    @functools.partial(
        pl.kernel,
        mesh=mesh,
        compiler_params=pltpu.CompilerParams(
            needs_layout_passes=False,
            use_tc_tiling_on_sc=False,
        ),
        out_type=jax.ShapeDtypeStruct((COLS, NW, DIM * IBLK), jnp.float32),
        scratch_types=[
            pltpu.VMEM((DIM * 16,), jnp.float32),
            pltpu.VMEM((NJB, JBLK * IBLK), jnp.int32),
            pltpu.VMEM((COLS, DIM * IBLK), jnp.float32),
            pltpu.SemaphoreType.DMA,
            pltpu.SemaphoreType.DMA,
        ],
    )
    def lookup(ids_hbm, table_hbm, out_hbm, table_v, idx_v, out_v, sem, sem2):
        wid = lax.axis_index("s") * NC + lax.axis_index("c")
        jsplit = 13
        cp_in0 = pltpu.async_copy(
            ids_hbm.at[pl.ds(0, jsplit), wid], idx_v.at[pl.ds(0, jsplit)], sem
        )
        cp_in1 = pltpu.async_copy(
            ids_hbm.at[pl.ds(jsplit, NJB - jsplit), wid],
            idx_v.at[pl.ds(jsplit, NJB - jsplit)],
            sem2,
        )
        pltpu.sync_copy(table_hbm, table_v)
        tvecs = [table_v[pl.ds(16 * d, 16)] for d in range(DIM)]
        half = COLS // 2

        def make_body(lo, hi):
            @plsc.parallel_loop(lo, hi, unroll=4)
            def body(j):
                jb = j // JBLK
                jm = (j % JBLK) * IBLK
                for k in range(IBLK // 16):
                    ids16 = idx_v[jb, pl.ds(jm + k * 16, 16)]
                    for d in range(DIM):
                        vals = tvecs[d].at[ids16].get(
                            mode="promise_in_bounds"
                        )
                        out_v[j, pl.ds(d * IBLK + k * 16, 16)] = vals

        cp_in0.wait()
        make_body(0, half)
        cp_in1.wait()
        cp = pltpu.async_copy(
            out_v.at[pl.ds(0, half)], out_hbm.at[pl.ds(0, half), wid], sem
        )
        make_body(half, COLS)
        cp.wait()
        pltpu.sync_copy(
            out_v.at[pl.ds(half, COLS - half)],
            out_hbm.at[pl.ds(half, COLS - half), wid],
        )

    return lookup


_lookup = _make_lookup()


def kernel(input_ids, embed_table):
    ids3 = (
        input_ids.astype(jnp.int32)
        .T.reshape(NJB, JBLK, NW, IBLK)
        .transpose(0, 2, 1, 3)
        .reshape(NJB, NW, JBLK * IBLK)
    )
    tpad = jnp.concatenate(
        [embed_table.T, jnp.zeros((DIM, 16 - NUM_EMB), jnp.float32)], axis=1
    ).reshape(-1)
    b = _lookup(ids3, tpad)
    return (
        b.reshape(COLS, NW, DIM, IBLK)
        .transpose(1, 3, 0, 2)
        .reshape(ROWS, COLS, DIM)
    )
